# baseline (speedup 1.0000x reference)
import functools

import jax
import jax.numpy as jnp
from jax import lax
from jax.experimental import pallas as pl
from jax.experimental.pallas import tpu as pltpu, tpu_sc as plsc

D = 1536
N_BINS = 16


_BT = 1024


def _dense_body(avg_ref, per_ref, w_ref, b_ref, o_ref):
    xa = jnp.broadcast_to(avg_ref[:], (_BT, N_BINS))
    xp = jnp.broadcast_to(per_ref[:], (_BT, N_BINS))
    cen = lax.broadcasted_iota(jnp.int32, (_BT, N_BINS), 1).astype(jnp.float32) * (
        1.0 / (N_BINS - 1)
    )
    za = (xa - cen) * float(N_BINS)
    zp = (xp - cen) * float(N_BINS)
    f = jnp.concatenate([jnp.exp(-za * za), jnp.exp(-zp * zp)], axis=1)
    o_ref[:] = (
        lax.dot_general(
            f,
            w_ref[:],
            (((1,), (0,)), ((), ())),
            preferred_element_type=jnp.float32,
            precision=lax.Precision.HIGHEST,
        )
        + b_ref[:]
    )


def _dense_part(avg, per, w, b, n):
    grid = n // _BT
    return pl.pallas_call(
        _dense_body,
        grid=(grid,),
        in_specs=[
            pl.BlockSpec((_BT, 1), lambda i: (i, 0)),
            pl.BlockSpec((_BT, 1), lambda i: (i, 0)),
            pl.BlockSpec((2 * N_BINS, D), lambda i: (0, 0)),
            pl.BlockSpec((1, D), lambda i: (0, 0)),
        ],
        out_specs=pl.BlockSpec((_BT, D), lambda i: (i, 0)),
        out_shape=jax.ShapeDtypeStruct((n, D), jnp.float32),
    )(avg, per, w, b)


_T = 16
_NW = 32
_NGRP = 20


def _add_group(acc, buf):
    def row(r, carry):
        def col(j, c2):
            sl = pl.ds(pl.multiple_of(j * 16, 16), 16)
            acc[r, sl] = acc[r, sl] + buf[r, sl]
            return c2

        return lax.fori_loop(0, D // 16, col, carry)

    lax.fori_loop(0, _T, row, 0)


def _add_func(acc, fbuf):
    def row(q, carry):
        r = q // 8
        k = q % 8

        def col(m, c2):
            asl = pl.ds(pl.multiple_of(k * 192 + m * 16, 16), 16)
            fsl = pl.ds(pl.multiple_of(m * 16, 16), 16)
            acc[r, asl] = acc[r, asl] + fbuf[q, fsl]
            return c2

        return lax.fori_loop(0, 192 // 16, col, carry)

    lax.fori_loop(0, 8 * _T, row, 0)


def _sc_body(struct_t, seq_t, ss8_t, sasa_t, res_t, func_t, idx_pack, fidx,
             dense, out, idxs_v, fidx_v, acc, buf0, buf1, fbuf,
             sem0, sem1, sem2):
    c = lax.axis_index("c")
    s = lax.axis_index("s")
    wid = s * 2 + c
    tables = [struct_t, seq_t, ss8_t, sasa_t] + [res_t] * 16
    bufs = [buf0, buf1]
    sems = [sem0, sem1]

    def chunk(ci, carry):
        cg = wid * (8192 // _NW // _T) + ci
        base = cg * _T
        pltpu.sync_copy(idx_pack.at[pl.ds(cg * _NGRP * _T, _NGRP * _T)], idxs_v)
        pltpu.sync_copy(fidx.at[pl.ds(base * 8, 8 * _T)], fidx_v)
        pltpu.sync_copy(dense.at[pl.ds(base, _T), :], acc)
        fcp = pltpu.async_copy(func_t.at[fidx_v], fbuf, sem2)
        handles = [None, None]
        for g in range(_NGRP):
            b = g % 2
            handles[b] = pltpu.async_copy(
                tables[g].at[idxs_v.at[pl.ds(g * _T, _T)]], bufs[b], sems[b]
            )
            if g >= 1:
                handles[1 - b].wait()
                _add_group(acc, bufs[1 - b])
        handles[(_NGRP - 1) % 2].wait()
        _add_group(acc, bufs[(_NGRP - 1) % 2])
        fcp.wait()
        _add_func(acc, fbuf)
        pltpu.sync_copy(acc, out.at[pl.ds(base, _T), :])
        return carry

    lax.fori_loop(0, 8192 // _NW // _T, chunk, 0)


def _sc_gather(struct_t, seq_t, ss8_t, sasa_t, res_t, func_t, idx_pack, fidx,
               dense, n):
    mesh = plsc.VectorSubcoreMesh(
        core_axis_name="c", subcore_axis_name="s", num_cores=2, num_subcores=16
    )
    fn = pl.kernel(
        _sc_body,
        out_type=jax.ShapeDtypeStruct((n, D), jnp.float32),
        mesh=mesh,
        scratch_types=[
            pltpu.VMEM((_NGRP * _T,), jnp.int32),
            pltpu.VMEM((8 * _T,), jnp.int32),
            pltpu.VMEM((_T, D), jnp.float32),
            pltpu.VMEM((_T, D), jnp.float32),
            pltpu.VMEM((_T, D), jnp.float32),
            pltpu.VMEM((8 * _T, 256), jnp.float32),
            pltpu.SemaphoreType.DMA,
            pltpu.SemaphoreType.DMA,
            pltpu.SemaphoreType.DMA,
        ],
    )
    return fn(struct_t, seq_t, ss8_t, sasa_t, res_t, func_t, idx_pack, fidx, dense)


def kernel(sequence_tokens, structure_tokens, average_plddt, per_res_plddt,
           ss8_tokens, sasa_tokens, function_tokens, residue_annotation_tokens,
           seq_table, struct_table, ss8_table, sasa_table, func_tables,
           residue_table, plddt_W, plddt_b, perres_W, perres_b):
    B, L = sequence_tokens.shape
    n = B * L

    wc = jnp.concatenate([plddt_W, perres_W], axis=0)
    bc = (plddt_b + perres_b).reshape(1, D)
    dense = _dense_part(
        average_plddt.reshape(n, 1), per_res_plddt.reshape(n, 1), wc, bc, n
    )

    res_idx = residue_annotation_tokens.reshape(n, 16)
    idx_pack = jnp.concatenate(
        [
            structure_tokens.reshape(1, n),
            sequence_tokens.reshape(1, n),
            ss8_tokens.reshape(1, n),
            sasa_tokens.reshape(1, n),
            res_idx.T,
        ],
        axis=0,
    ).astype(jnp.int32)
    idx_pack = (
        idx_pack.reshape(_NGRP, n // _T, _T)
        .transpose(1, 0, 2)
        .reshape(n // _T * _NGRP * _T)
    )
    fidx = (
        function_tokens.reshape(n, 8) + jnp.arange(8, dtype=jnp.int32) * 260
    ).reshape(n * 8).astype(jnp.int32)
    func_flat = jnp.pad(func_tables.reshape(8 * 260, D // 8), ((0, 0), (0, 64)))

    out = _sc_gather(
        struct_table, seq_table, ss8_table, sasa_table, residue_table,
        func_flat, idx_pack, fidx, dense, n,
    )
    return out.reshape(B, L, D)
